# Initial kernel scaffold; baseline (speedup 1.0000x reference)
import functools

import jax
import jax.numpy as jnp
from jax import lax
from jax.experimental import pallas as pl
from jax.experimental.pallas import tpu as pltpu
from jax.experimental.pallas import tpu_sc as plsc

_D = 128
_B = 4096
_X_MAX = 100.0
_NC, _NS, _L = 2, 16, 16
_NW = _NC * _NS
_BPW = _B // _NW
_GROUPS = _BPW // _L

_LN2 = 0.6931471805599453
_LN_XMAX = 4.605170185988092


def _ln(x):
  xi = plsc.bitcast(x, jnp.int32)
  e = lax.shift_right_logical(xi, 23) - 127
  m = plsc.bitcast((xi & 0x7FFFFF) | 0x3F800000, jnp.float32)
  big = m > 1.4142135623730951
  m = jnp.where(big, m * 0.5, m)
  e = jnp.where(big, e + 1, e)
  s = (m - 1.0) / (m + 1.0)
  t = s * s
  p = s * (2.0 + t * (2.0 / 3.0 + t * (2.0 / 5.0 + t * (2.0 / 7.0 + t * (2.0 / 9.0)))))
  return e.astype(jnp.float32) * _LN2 + p


@functools.partial(
    pl.kernel,
    out_type=jax.ShapeDtypeStruct((_NW, 5, _L), jnp.float32),
    mesh=plsc.VectorSubcoreMesh(
        core_axis_name="c", subcore_axis_name="s", num_cores=_NC,
        num_subcores=_NS),
    scratch_types=[
        pltpu.VMEM((_BPW,), jnp.int32),
        pltpu.VMEM((_BPW,), jnp.int32),
        pltpu.VMEM((_BPW,), jnp.float32),
        pltpu.VMEM((_BPW,), jnp.float32),
        pltpu.VMEM((_BPW,), jnp.float32),
        pltpu.VMEM((_BPW, _D), jnp.float32),
        pltpu.VMEM((_BPW, _D), jnp.float32),
        pltpu.VMEM((5, _L), jnp.float32),
        pltpu.SemaphoreType.DMA,
        pltpu.SemaphoreType.DMA,
        pltpu.SemaphoreType.DMA,
        pltpu.SemaphoreType.DMA,
    ],
)
def _glove_partials(fi_hbm, ci_hbm, cc_hbm, ftab_hbm, ctab_hbm, fb_hbm,
                    cb_hbm, out_hbm, idx_f, idx_c, cc_v, fb_v, cb_v, rows_f,
                    rows_c, part_v, sem_f, sem_c, sem_fb, sem_cb):
  wid = lax.axis_index("s") * _NC + lax.axis_index("c")
  base = wid * _BPW

  pltpu.sync_copy(fi_hbm.at[pl.ds(base, _BPW)], idx_f)
  pltpu.sync_copy(ci_hbm.at[pl.ds(base, _BPW)], idx_c)
  pltpu.sync_copy(cc_hbm.at[pl.ds(base, _BPW)], cc_v)

  cp_f = pltpu.async_copy(ftab_hbm.at[idx_f], rows_f, sem_f)
  cp_c = pltpu.async_copy(ctab_hbm.at[idx_c], rows_c, sem_c)
  cp_fb = pltpu.async_copy(fb_hbm.at[idx_f], fb_v, sem_fb)
  cp_cb = pltpu.async_copy(cb_hbm.at[idx_c], cb_v, sem_cb)
  cp_f.wait()
  cp_c.wait()
  cp_fb.wait()
  cp_cb.wait()

  lane = lax.iota(jnp.int32, _L)
  zero = jnp.zeros((_L,), jnp.float32)
  acc1 = acc2 = acc3 = acc4 = acc5 = zero

  for g in range(_GROUPS):
    rows = g * _L + lane

    def dot_step(d, acc):
      col = (d + lane) & (_D - 1)
      f = plsc.load_gather(rows_f, [rows, col])
      c = plsc.load_gather(rows_c, [rows, col])
      return acc + f * c

    ep = lax.fori_loop(0, _D, dot_step, zero, unroll=4)

    sl = pl.ds(g * _L, _L)
    cc = cc_v[sl]
    lc = _ln(cc)
    wf = jnp.minimum(jnp.exp(0.75 * (lc - _LN_XMAX)), 1.0)
    a = ep + lc
    b = fb_v[sl] + cb_v[sl]
    acc1 = acc1 + wf * a * a
    acc2 = acc2 + wf * a
    acc3 = acc3 + wf
    acc4 = acc4 + b
    acc5 = acc5 + b * b

  part_v[0, :] = acc1
  part_v[1, :] = acc2
  part_v[2, :] = acc3
  part_v[3, :] = acc4
  part_v[4, :] = acc5
  pltpu.sync_copy(part_v, out_hbm.at[wid])


def _combine(p_ref, o_ref):
  s1 = jnp.sum(p_ref[:, 0, :])
  s2 = jnp.sum(p_ref[:, 1, :])
  s3 = jnp.sum(p_ref[:, 2, :])
  s4 = jnp.sum(p_ref[:, 3, :])
  s5 = jnp.sum(p_ref[:, 4, :])
  o_ref[0, 0] = (_B * s1 + 2.0 * s2 * s4 + s3 * s5) / (float(_B) * float(_B))


def kernel(focal_input, context_input, coocurrence_count, focal_table,
           context_table, focal_bias_table, context_bias_table):
  parts = _glove_partials(
      focal_input, context_input, coocurrence_count, focal_table,
      context_table, focal_bias_table.reshape(-1),
      context_bias_table.reshape(-1))
  out = pl.pallas_call(
      _combine,
      out_shape=jax.ShapeDtypeStruct((1, 1), jnp.float32),
  )(parts)
  return out[0, 0]


# speedup vs baseline: 2.4339x; 2.4339x over previous
import functools

import jax
import jax.numpy as jnp
from jax import lax
from jax.experimental import pallas as pl
from jax.experimental.pallas import tpu as pltpu
from jax.experimental.pallas import tpu_sc as plsc

_D = 128
_B = 4096
_X_MAX = 100.0
_NC, _NS, _L = 2, 16, 16
_NW = _NC * _NS
_BPW = _B // _NW
_GROUPS = _BPW // _L

_LN2 = 0.6931471805599453
_LN_XMAX = 4.605170185988092


def _ln(x):
  xi = plsc.bitcast(x, jnp.int32)
  e = lax.shift_right_logical(xi, 23) - 127
  m = plsc.bitcast((xi & 0x7FFFFF) | 0x3F800000, jnp.float32)
  big = m > 1.4142135623730951
  m = jnp.where(big, m * 0.5, m)
  e = jnp.where(big, e + 1, e)
  s = (m - 1.0) / (m + 1.0)
  t = s * s
  p = s * (2.0 + t * (2.0 / 3.0 + t * (2.0 / 5.0 + t * (2.0 / 7.0 + t * (2.0 / 9.0)))))
  return e.astype(jnp.float32) * _LN2 + p


@functools.partial(
    pl.kernel,
    out_type=jax.ShapeDtypeStruct((_NW, 5, _L), jnp.float32),
    mesh=plsc.VectorSubcoreMesh(
        core_axis_name="c", subcore_axis_name="s", num_cores=_NC,
        num_subcores=_NS),
    compiler_params=pltpu.CompilerParams(needs_layout_passes=False),
    scratch_types=[
        pltpu.VMEM((_BPW,), jnp.int32),
        pltpu.VMEM((_BPW,), jnp.int32),
        pltpu.VMEM((_BPW,), jnp.float32),
        pltpu.VMEM((_BPW,), jnp.float32),
        pltpu.VMEM((_BPW,), jnp.float32),
        pltpu.VMEM((_BPW, _D), jnp.float32),
        pltpu.VMEM((_BPW, _D), jnp.float32),
        pltpu.VMEM((5, _L), jnp.float32),
        pltpu.SemaphoreType.DMA,
        pltpu.SemaphoreType.DMA,
        pltpu.SemaphoreType.DMA,
        pltpu.SemaphoreType.DMA,
    ],
)
def _glove_partials(fi_hbm, ci_hbm, cc_hbm, ftab_hbm, ctab_hbm, fb_hbm,
                    cb_hbm, out_hbm, idx_f, idx_c, cc_v, fb_v, cb_v, rows_f,
                    rows_c, part_v, sem_f, sem_c, sem_fb, sem_cb):
  wid = lax.axis_index("s") * _NC + lax.axis_index("c")
  base = wid * _BPW

  pltpu.sync_copy(fi_hbm.at[pl.ds(base, _BPW)], idx_f)
  pltpu.sync_copy(ci_hbm.at[pl.ds(base, _BPW)], idx_c)
  pltpu.sync_copy(cc_hbm.at[pl.ds(base, _BPW)], cc_v)

  cp_f = pltpu.async_copy(ftab_hbm.at[idx_f], rows_f, sem_f)
  cp_c = pltpu.async_copy(ctab_hbm.at[idx_c], rows_c, sem_c)
  cp_fb = pltpu.async_copy(fb_hbm.at[idx_f], fb_v, sem_fb)
  cp_cb = pltpu.async_copy(cb_hbm.at[idx_c], cb_v, sem_cb)
  cp_f.wait()
  cp_c.wait()
  cp_fb.wait()
  cp_cb.wait()

  lane = lax.iota(jnp.int32, _L)
  zero = jnp.zeros((_L,), jnp.float32)
  acc1 = acc2 = acc3 = acc4 = acc5 = zero

  for g in range(_GROUPS):
    rows = g * _L + lane

    def dot_step(d, acc):
      col = (d + lane) & (_D - 1)
      f = plsc.load_gather(rows_f, [rows, col])
      c = plsc.load_gather(rows_c, [rows, col])
      return acc + f * c

    ep = lax.fori_loop(0, _D, dot_step, zero, unroll=4)

    sl = pl.ds(g * _L, _L)
    cc = cc_v[sl]
    lc = _ln(cc)
    wf = jnp.minimum(jnp.exp(0.75 * (lc - _LN_XMAX)), 1.0)
    a = ep + lc
    b = fb_v[sl] + cb_v[sl]
    acc1 = acc1 + wf * a * a
    acc2 = acc2 + wf * a
    acc3 = acc3 + wf
    acc4 = acc4 + b
    acc5 = acc5 + b * b

  part_v[0, :] = acc1
  part_v[1, :] = acc2
  part_v[2, :] = acc3
  part_v[3, :] = acc4
  part_v[4, :] = acc5
  pltpu.sync_copy(part_v, out_hbm.at[wid])


def _combine(p_ref, o_ref):
  s1 = jnp.sum(p_ref[:, 0, :])
  s2 = jnp.sum(p_ref[:, 1, :])
  s3 = jnp.sum(p_ref[:, 2, :])
  s4 = jnp.sum(p_ref[:, 3, :])
  s5 = jnp.sum(p_ref[:, 4, :])
  loss = (_B * s1 + 2.0 * s2 * s4 + s3 * s5) / (float(_B) * float(_B))
  o_ref[...] = jnp.broadcast_to(loss, (1, 1))


def kernel(focal_input, context_input, coocurrence_count, focal_table,
           context_table, focal_bias_table, context_bias_table):
  parts = _glove_partials(
      focal_input, context_input, coocurrence_count, focal_table,
      context_table, focal_bias_table.reshape(-1),
      context_bias_table.reshape(-1))
  out = pl.pallas_call(
      _combine,
      out_shape=jax.ShapeDtypeStruct((1, 1), jnp.float32),
  )(parts)
  return out[0, 0]


# speedup vs baseline: 2.5496x; 1.0475x over previous
import functools

import jax
import jax.numpy as jnp
from jax import lax
from jax.experimental import pallas as pl
from jax.experimental.pallas import tpu as pltpu
from jax.experimental.pallas import tpu_sc as plsc

_D = 128
_B = 4096
_V = 100000
_NC, _NS, _L = 2, 16, 16
_NW = _NC * _NS
_BPW = _B // _NW
_CH = 4
_RPC = _BPW // _CH
_GROUPS = _BPW // _L
_GPC = _GROUPS // _CH

_LN2 = 0.6931471805599453
_LN_XMAX = 4.605170185988092


def _ln(x):
  xi = plsc.bitcast(x, jnp.int32)
  e = lax.shift_right_logical(xi, 23) - 127
  m = plsc.bitcast((xi & 0x7FFFFF) | 0x3F800000, jnp.float32)
  big = m > 1.4142135623730951
  m = jnp.where(big, m * 0.5, m)
  e = jnp.where(big, e + 1, e)
  s = (m - 1.0) / (m + 1.0)
  t = s * s
  p = s * (2.0 + t * (2.0 / 3.0 + t * (2.0 / 5.0 + t * (2.0 / 7.0 + t * (2.0 / 9.0)))))
  return e.astype(jnp.float32) * _LN2 + p


@functools.partial(
    pl.kernel,
    out_type=jax.ShapeDtypeStruct((_NW, 5, _L), jnp.float32),
    mesh=plsc.VectorSubcoreMesh(
        core_axis_name="c", subcore_axis_name="s", num_cores=_NC,
        num_subcores=_NS),
    compiler_params=pltpu.CompilerParams(needs_layout_passes=False),
    scratch_types=[
        pltpu.VMEM((_BPW,), jnp.int32),
        pltpu.VMEM((_BPW,), jnp.int32),
        pltpu.VMEM((_BPW,), jnp.int32),
        pltpu.VMEM((_BPW,), jnp.float32),
        pltpu.VMEM((_BPW,), jnp.float32),
        pltpu.VMEM((_BPW,), jnp.float32),
        pltpu.VMEM((_BPW, _D), jnp.float32),
        pltpu.VMEM((_BPW, _D), jnp.float32),
        pltpu.VMEM((5, _L), jnp.float32),
        pltpu.SemaphoreType.DMA,
        pltpu.SemaphoreType.DMA,
        pltpu.SemaphoreType.DMA,
        pltpu.SemaphoreType.DMA,
        pltpu.SemaphoreType.DMA,
        pltpu.SemaphoreType.DMA,
        pltpu.SemaphoreType.DMA,
        pltpu.SemaphoreType.DMA,
        pltpu.SemaphoreType.DMA,
    ],
)
def _glove_partials(fi_hbm, ci_hbm, cc_hbm, ftab_hbm, ctab_hbm, bias_hbm,
                    out_hbm, idx_f, idx_c, idx_cb, cc_v, fb_v, cb_v, rows_f,
                    rows_c, part_v, sem_if, sem_ic, sem_cc, sem_fb, sem_cb,
                    sem_r0, sem_r1, sem_r2, sem_r3):
  wid = lax.axis_index("s") * _NC + lax.axis_index("c")
  base = wid * _BPW
  sem_r = (sem_r0, sem_r1, sem_r2, sem_r3)

  cp_if = pltpu.async_copy(fi_hbm.at[pl.ds(base, _BPW)], idx_f, sem_if)
  cp_ic = pltpu.async_copy(ci_hbm.at[pl.ds(base, _BPW)], idx_c, sem_ic)
  cp_cc = pltpu.async_copy(cc_hbm.at[pl.ds(base, _BPW)], cc_v, sem_cc)

  cp_if.wait()
  row_cps = []
  sl0 = pl.ds(0, _RPC)
  row_cps.append((pltpu.async_copy(
      ftab_hbm.at[idx_f.at[sl0]], rows_f.at[sl0], sem_r[0]), None))
  cp_fb = pltpu.async_copy(bias_hbm.at[idx_f], fb_v, sem_fb)

  cp_ic.wait()
  cp_c0 = pltpu.async_copy(
      ctab_hbm.at[idx_c.at[sl0]], rows_c.at[sl0], sem_r[0])
  row_cps[0] = (row_cps[0][0], cp_c0)
  for ch in range(1, _CH):
    sl = pl.ds(ch * _RPC, _RPC)
    cpf = pltpu.async_copy(ftab_hbm.at[idx_f.at[sl]], rows_f.at[sl],
                           sem_r[ch])
    cpc = pltpu.async_copy(ctab_hbm.at[idx_c.at[sl]], rows_c.at[sl],
                           sem_r[ch])
    row_cps.append((cpf, cpc))

  for k in range(_BPW // _L):
    ksl = pl.ds(k * _L, _L)
    idx_cb[ksl] = idx_c[ksl] + _V
  cp_cb = pltpu.async_copy(bias_hbm.at[idx_cb], cb_v, sem_cb)

  cp_cc.wait()
  cp_fb.wait()
  cp_cb.wait()

  lane = lax.iota(jnp.int32, _L)
  zero = jnp.zeros((_L,), jnp.float32)
  acc1 = acc2 = acc3 = acc4 = acc5 = zero

  for ch in range(_CH):
    row_cps[ch][0].wait()
    row_cps[ch][1].wait()
    for g in range(ch * _GPC, (ch + 1) * _GPC):
      rows = g * _L + lane

      def dot_step(d, st):
        acc, col = st
        f = plsc.load_gather(rows_f, [rows, col])
        c = plsc.load_gather(rows_c, [rows, col])
        return acc + f * c, (col + 1) & (_D - 1)

      ep, _ = lax.fori_loop(0, _D, dot_step, (zero, lane), unroll=8)

      sl = pl.ds(g * _L, _L)
      cc = cc_v[sl]
      lc = _ln(cc)
      wf = jnp.minimum(jnp.exp(0.75 * (lc - _LN_XMAX)), 1.0)
      a = ep + lc
      b = fb_v[sl] + cb_v[sl]
      acc1 = acc1 + wf * a * a
      acc2 = acc2 + wf * a
      acc3 = acc3 + wf
      acc4 = acc4 + b
      acc5 = acc5 + b * b

  part_v[0, :] = acc1
  part_v[1, :] = acc2
  part_v[2, :] = acc3
  part_v[3, :] = acc4
  part_v[4, :] = acc5
  pltpu.sync_copy(part_v, out_hbm.at[wid])


def _combine(p_ref, o_ref):
  s1 = jnp.sum(p_ref[:, 0, :])
  s2 = jnp.sum(p_ref[:, 1, :])
  s3 = jnp.sum(p_ref[:, 2, :])
  s4 = jnp.sum(p_ref[:, 3, :])
  s5 = jnp.sum(p_ref[:, 4, :])
  loss = (_B * s1 + 2.0 * s2 * s4 + s3 * s5) / (float(_B) * float(_B))
  o_ref[...] = jnp.broadcast_to(loss, (1, 1))


def kernel(focal_input, context_input, coocurrence_count, focal_table,
           context_table, focal_bias_table, context_bias_table):
  bias_cat = jnp.concatenate(
      [focal_bias_table.reshape(-1), context_bias_table.reshape(-1)])
  parts = _glove_partials(
      focal_input, context_input, coocurrence_count, focal_table,
      context_table, bias_cat)
  out = pl.pallas_call(
      _combine,
      out_shape=jax.ShapeDtypeStruct((1, 1), jnp.float32),
  )(parts)
  return out[0, 0]


# speedup vs baseline: 2.5743x; 1.0097x over previous
import functools

import jax
import jax.numpy as jnp
from jax import lax
from jax.experimental import pallas as pl
from jax.experimental.pallas import tpu as pltpu
from jax.experimental.pallas import tpu_sc as plsc

_D = 128
_B = 4096
_V = 100000
_NC, _NS, _L = 2, 16, 16
_NW = _NC * _NS
_BPW = _B // _NW
_CH = 2
_RPC = _BPW // _CH
_GROUPS = _BPW // _L
_GPC = _GROUPS // _CH

_LN2 = 0.6931471805599453
_LN_XMAX = 4.605170185988092


def _ln(x):
  xi = plsc.bitcast(x, jnp.int32)
  e = lax.shift_right_logical(xi, 23) - 127
  m = plsc.bitcast((xi & 0x7FFFFF) | 0x3F800000, jnp.float32)
  big = m > 1.4142135623730951
  m = jnp.where(big, m * 0.5, m)
  e = jnp.where(big, e + 1, e)
  s = (m - 1.0) / (m + 1.0)
  t = s * s
  p = s * (2.0 + t * (2.0 / 3.0 + t * (2.0 / 5.0 + t * (2.0 / 7.0 + t * (2.0 / 9.0)))))
  return e.astype(jnp.float32) * _LN2 + p


@functools.partial(
    pl.kernel,
    out_type=jax.ShapeDtypeStruct((_NW, 5, _L), jnp.float32),
    mesh=plsc.VectorSubcoreMesh(
        core_axis_name="c", subcore_axis_name="s", num_cores=_NC,
        num_subcores=_NS),
    compiler_params=pltpu.CompilerParams(needs_layout_passes=False),
    scratch_types=[
        pltpu.VMEM((_BPW,), jnp.int32),
        pltpu.VMEM((_BPW,), jnp.int32),
        pltpu.VMEM((_BPW,), jnp.int32),
        pltpu.VMEM((_BPW,), jnp.float32),
        pltpu.VMEM((_BPW,), jnp.float32),
        pltpu.VMEM((_BPW,), jnp.float32),
        pltpu.VMEM((_BPW, _D), jnp.float32),
        pltpu.VMEM((_BPW, _D), jnp.float32),
        pltpu.VMEM((5, _L), jnp.float32),
        pltpu.SemaphoreType.DMA,
        pltpu.SemaphoreType.DMA,
        pltpu.SemaphoreType.DMA,
        pltpu.SemaphoreType.DMA,
        pltpu.SemaphoreType.DMA,
        pltpu.SemaphoreType.DMA,
        pltpu.SemaphoreType.DMA,
    ],
)
def _glove_partials(fi_hbm, ci_hbm, cc_hbm, ftab_hbm, ctab_hbm, bias_hbm,
                    out_hbm, idx_f, idx_c, idx_cb, cc_v, fb_v, cb_v, rows_f,
                    rows_c, part_v, sem_if, sem_ic, sem_cc, sem_fb, sem_cb,
                    sem_r0, sem_r1):
  wid = lax.axis_index("s") * _NC + lax.axis_index("c")
  base = wid * _BPW
  sem_r = (sem_r0, sem_r1)

  cp_if = pltpu.async_copy(fi_hbm.at[pl.ds(base, _BPW)], idx_f, sem_if)
  cp_ic = pltpu.async_copy(ci_hbm.at[pl.ds(base, _BPW)], idx_c, sem_ic)
  cp_cc = pltpu.async_copy(cc_hbm.at[pl.ds(base, _BPW)], cc_v, sem_cc)

  cp_if.wait()
  sl0 = pl.ds(0, _RPC)
  sl1 = pl.ds(_RPC, _RPC)
  cp_f0 = pltpu.async_copy(ftab_hbm.at[idx_f.at[sl0]], rows_f.at[sl0],
                           sem_r[0])
  cp_fb = pltpu.async_copy(bias_hbm.at[idx_f], fb_v, sem_fb)

  cp_ic.wait()
  cp_c0 = pltpu.async_copy(ctab_hbm.at[idx_c.at[sl0]], rows_c.at[sl0],
                           sem_r[0])
  cp_f1 = pltpu.async_copy(ftab_hbm.at[idx_f.at[sl1]], rows_f.at[sl1],
                           sem_r[1])
  cp_c1 = pltpu.async_copy(ctab_hbm.at[idx_c.at[sl1]], rows_c.at[sl1],
                           sem_r[1])

  def mk_cb(k, _):
    ksl = pl.ds(k * _L, _L)
    idx_cb[ksl] = idx_c[ksl] + _V
    return 0

  lax.fori_loop(0, _BPW // _L, mk_cb, 0)
  cp_cb = pltpu.async_copy(bias_hbm.at[idx_cb], cb_v, sem_cb)

  cp_cc.wait()
  cp_fb.wait()
  cp_cb.wait()

  lane = lax.iota(jnp.int32, _L)
  zero = jnp.zeros((_L,), jnp.float32)
  accs = (zero, zero, zero, zero, zero)
  waits = ((cp_f0, cp_c0), (cp_f1, cp_c1))

  for ch in range(_CH):
    waits[ch][0].wait()
    waits[ch][1].wait()

    def group(g, accs):
      acc1, acc2, acc3, acc4, acc5 = accs
      rows = g * _L + lane

      def dot_step(d, st):
        acc, col = st
        f = plsc.load_gather(rows_f, [rows, col])
        c = plsc.load_gather(rows_c, [rows, col])
        return acc + f * c, (col + 1) & (_D - 1)

      ep, _ = lax.fori_loop(0, _D, dot_step, (zero, lane), unroll=8)

      sl = pl.ds(g * _L, _L)
      cc = cc_v[sl]
      lc = _ln(cc)
      wf = jnp.minimum(jnp.exp(0.75 * (lc - _LN_XMAX)), 1.0)
      a = ep + lc
      b = fb_v[sl] + cb_v[sl]
      return (acc1 + wf * a * a, acc2 + wf * a, acc3 + wf, acc4 + b,
              acc5 + b * b)

    accs = lax.fori_loop(ch * _GPC, (ch + 1) * _GPC, group, accs)

  for k in range(5):
    part_v[k, :] = accs[k]
  pltpu.sync_copy(part_v, out_hbm.at[wid])


def _combine(p_ref, o_ref):
  s1 = jnp.sum(p_ref[:, 0, :])
  s2 = jnp.sum(p_ref[:, 1, :])
  s3 = jnp.sum(p_ref[:, 2, :])
  s4 = jnp.sum(p_ref[:, 3, :])
  s5 = jnp.sum(p_ref[:, 4, :])
  loss = (_B * s1 + 2.0 * s2 * s4 + s3 * s5) / (float(_B) * float(_B))
  o_ref[...] = jnp.broadcast_to(loss, (1, 1))


def kernel(focal_input, context_input, coocurrence_count, focal_table,
           context_table, focal_bias_table, context_bias_table):
  bias_cat = jnp.concatenate(
      [focal_bias_table.reshape(-1), context_bias_table.reshape(-1)])
  parts = _glove_partials(
      focal_input, context_input, coocurrence_count, focal_table,
      context_table, bias_cat)
  out = pl.pallas_call(
      _combine,
      out_shape=jax.ShapeDtypeStruct((1, 1), jnp.float32),
  )(parts)
  return out[0, 0]
